# baseline (speedup 1.0000x reference)
import functools

import jax
import jax.numpy as jnp
from jax import lax
from jax.experimental import pallas as pl
from jax.experimental.pallas import tpu as pltpu
from jax.experimental.pallas import tpu_sc as plsc

N = 10000
E = 320000
D = 128
G = 64
OUT = 64

NC = 2
NS = 16
NW = NC * NS
E_PER_TILE = E // NW
CHUNK = 80
NCHUNK = E_PER_TILE // CHUNK
RBLK = 80
NRBLK = N // RBLK

NBUF = 4
IBUF = 2 * NBUF
K = 2
L = IBUF - NBUF + K
ZROWS = 16
GROUPS = NCHUNK // IBUF
TAIL = NCHUNK - GROUPS * IBUF


@functools.cache
def _make_sc_aggregate():
    mesh = plsc.VectorSubcoreMesh(core_axis_name="c", subcore_axis_name="s",
                                  num_cores=NC, num_subcores=NS)
    return pl.kernel(
        _sc_aggregate_body,
        out_type=jax.ShapeDtypeStruct((2 * N, D), jnp.float32),
        mesh=mesh,
        scratch_types=[
            pltpu.VMEM((ZROWS, D), jnp.float32),
            pltpu.VMEM_SHARED((N, D), jnp.float32),
        ]
        + [pltpu.VMEM((CHUNK, D), jnp.float32) for _ in range(NBUF)]
        + [pltpu.VMEM((CHUNK,), jnp.int32) for _ in range(IBUF)]
        + [pltpu.VMEM((CHUNK,), jnp.int32) for _ in range(IBUF)]
        + [pltpu.SemaphoreType.DMA for _ in range(2 * NBUF + IBUF)],
    )


def _sc_aggregate_body(h_hbm, ei_hbm, out_hbm, zbuf, acc, *rest):
    rows = rest[:NBUF]
    isl_s = rest[NBUF:NBUF + IBUF]
    isl_d = rest[NBUF + IBUF:NBUF + 2 * IBUF]
    sem_g = rest[NBUF + 2 * IBUF:2 * NBUF + 2 * IBUF]
    sem_s = rest[2 * NBUF + 2 * IBUF:3 * NBUF + 2 * IBUF]
    sem_i = rest[3 * NBUF + 2 * IBUF:]
    cid = lax.axis_index("c")
    sid = lax.axis_index("s")
    wid = sid * NC + cid
    base = wid * E_PER_TILE

    def load_idx(j, q):
        pltpu.async_copy(ei_hbm.at[pl.ds(base + j * CHUNK, CHUNK)],
                         isl_s[q], sem_i[q])
        pltpu.async_copy(ei_hbm.at[pl.ds(E + base + j * CHUNK, CHUNK)],
                         isl_d[q], sem_i[q])

    def wait_idx(j, q):
        pltpu.make_async_copy(ei_hbm.at[pl.ds(base + j * CHUNK, CHUNK)],
                              isl_s[q], sem_i[q]).wait()
        pltpu.make_async_copy(ei_hbm.at[pl.ds(E + base + j * CHUNK, CHUNK)],
                              isl_d[q], sem_i[q]).wait()

    def gather(b, q):
        pltpu.async_copy(h_hbm.at[isl_s[q]], rows[b], sem_g[b])

    def wait_gather(b, q):
        pltpu.make_async_copy(h_hbm.at[isl_s[q]], rows[b], sem_g[b]).wait()

    def scatter(b, q):
        pltpu.async_copy(rows[b], acc.at[isl_d[q]], sem_s[b], add=True)

    def wait_scatter(b, q):
        pltpu.make_async_copy(rows[b], acc.at[isl_d[q]], sem_s[b]).wait()

    for q in range(L):
        load_idx(q, q)

    z16 = jnp.zeros((16,), jnp.float32)

    def _zrow(i, carry):
        for j in range(D // 16):
            zbuf[i, pl.ds(j * 16, 16)] = z16
        return carry

    lax.fori_loop(0, ZROWS, _zrow, 0)
    nzblk = N // ZROWS

    def _zblk(k, carry):
        blk = sid + k * NS

        @pl.when(blk < nzblk)
        def _():
            pltpu.sync_copy(zbuf, acc.at[pl.ds(blk * ZROWS, ZROWS)])
        return carry

    lax.fori_loop(0, pl.cdiv(nzblk, NS), _zblk, 0)

    for c in range(K):
        wait_idx(c, c)
        gather(c % NBUF, c)
    plsc.subcore_barrier()

    def _step(j, u, static):
        b = u % NBUF
        wait_gather(b, u)
        scatter(b, u)
        b2 = (u + K) % NBUF
        u2 = (u + K) % IBUF
        ud = (u + K - NBUF) % IBUF

        def _drain_prev():
            wait_scatter(b2, ud)

        def _prefetch():
            wait_idx(j + K, u2)
            gather(b2, u2)

        def _load():
            load_idx(j + L, (u + L) % IBUF)

        if static:
            if j + K < NCHUNK:
                if j >= NBUF - K:
                    _drain_prev()
                _prefetch()
            if j + L < NCHUNK:
                _load()
        else:
            def _both():
                pl.when(j >= NBUF - K)(_drain_prev)
                _prefetch()

            pl.when(j + K < NCHUNK)(_both)
            pl.when(j + L < NCHUNK)(_load)

    def _group(g, carry):
        for u in range(IBUF):
            _step(g * IBUF + u, u, False)
        return carry

    lax.fori_loop(0, GROUPS, _group, 0)

    for t in range(TAIL):
        j = GROUPS * IBUF + t
        _step(j, j % IBUF, True)

    for c in range(NCHUNK - NBUF, NCHUNK):
        wait_scatter(c % NBUF, c % IBUF)
    plsc.subcore_barrier()

    def _oblk(k, carry):
        blk = sid + k * NS

        @pl.when(blk < NRBLK)
        def _():
            pltpu.sync_copy(acc.at[pl.ds(blk * RBLK, RBLK)],
                            out_hbm.at[pl.ds(cid * N + blk * RBLK, RBLK)])
        return carry

    lax.fori_loop(0, pl.cdiv(NRBLK, NS), _oblk, 0)


BR = 2000
NB = N // BR


_PREC = lax.Precision.DEFAULT


def _tc_weights_body(batchf_ref, w_ref):
    onehot = (batchf_ref[...] ==
              lax.broadcasted_iota(jnp.int32, (N, G), 1).astype(jnp.float32)
              ).astype(jnp.float32)
    cnt = jnp.sum(onehot, axis=0, keepdims=True)
    inv = 1.0 / jnp.maximum(cnt, 1.0)
    w_ref[...] = lax.dot_general(onehot, inv, (((1,), (1,)), ((), ())),
                                 preferred_element_type=jnp.float32,
                                 precision=_PREC)


_tc_weights = pl.pallas_call(
    _tc_weights_body,
    out_shape=jax.ShapeDtypeStruct((N, 1), jnp.float32),
)


def _mlp(h, agg0, agg1, scale, w1, b1, w2, b2):
    u = h * scale + agg0 + agg1
    t = jnp.maximum(jnp.dot(u, w1, preferred_element_type=jnp.float32,
                            precision=_PREC) + b1, 0.0)
    return jnp.dot(t, w2, preferred_element_type=jnp.float32,
                   precision=_PREC) + b2


def _tc_layer_body(scale_ref, h_ref, agg_ref, w_ref,
                   w1_ref, b1_ref, w2_ref, b2_ref, out_h_ref, out_p_ref):
    i = pl.program_id(0)
    v = _mlp(h_ref[...], agg_ref[0], agg_ref[1], scale_ref[0, 0],
             w1_ref[...], b1_ref[...], w2_ref[...], b2_ref[...])
    out_h_ref[...] = v
    part = lax.dot_general(w_ref[...], v, (((0,), (0,)), ((), ())),
                           preferred_element_type=jnp.float32,
                           precision=_PREC)

    @pl.when(i == 0)
    def _():
        out_p_ref[...] = part

    @pl.when(i != 0)
    def _():
        out_p_ref[...] += part


_layer_in_specs = [
    pl.BlockSpec((1, 1), lambda i: (0, 0)),
    pl.BlockSpec((BR, D), lambda i: (i, 0)),
    pl.BlockSpec((2, BR, D), lambda i: (0, i, 0)),
    pl.BlockSpec((BR, 1), lambda i: (i, 0)),
    pl.BlockSpec((D, D), lambda i: (0, 0)),
    pl.BlockSpec((1, D), lambda i: (0, 0)),
    pl.BlockSpec((D, D), lambda i: (0, 0)),
    pl.BlockSpec((1, D), lambda i: (0, 0)),
]

_tc_layer = pl.pallas_call(
    _tc_layer_body,
    grid=(NB,),
    in_specs=_layer_in_specs,
    out_shape=[jax.ShapeDtypeStruct((N, D), jnp.float32),
               jax.ShapeDtypeStruct((1, D), jnp.float32)],
    out_specs=[pl.BlockSpec((BR, D), lambda i: (i, 0)),
               pl.BlockSpec((1, D), lambda i: (0, 0))],
)


def _tc_last_body(scale_ref, h_ref, agg_ref, w_ref,
                  w1_ref, b1_ref, w2_ref, b2_ref,
                  p1_ref, p2_ref, l1w_ref, l1b_ref, l2w_ref, l2b_ref,
                  out_ref, pacc_ref):
    i = pl.program_id(0)
    v = _mlp(h_ref[...], agg_ref[0], agg_ref[1], scale_ref[0, 0],
             w1_ref[...], b1_ref[...], w2_ref[...], b2_ref[...])
    part = lax.dot_general(w_ref[...], v, (((0,), (0,)), ((), ())),
                           preferred_element_type=jnp.float32,
                           precision=_PREC)

    @pl.when(i == 0)
    def _():
        pacc_ref[...] = part

    @pl.when(i != 0)
    def _():
        pacc_ref[...] += part

    @pl.when(i == NB - 1)
    def _():
        p = jnp.concatenate([p1_ref[...], p2_ref[...], pacc_ref[...]], axis=1)
        o = jnp.maximum(jnp.dot(p, l1w_ref[...],
                                preferred_element_type=jnp.float32,
                                precision=_PREC) + l1b_ref[...], 0.0)
        o = jnp.maximum(jnp.dot(o, l2w_ref[...],
                                preferred_element_type=jnp.float32,
                                precision=_PREC) + l2b_ref[...], 0.0)
        out_ref[...] = o


_tc_last = pl.pallas_call(
    _tc_last_body,
    grid=(NB,),
    in_specs=_layer_in_specs + [
        pl.BlockSpec((1, D), lambda i: (0, 0)),
        pl.BlockSpec((1, D), lambda i: (0, 0)),
        pl.BlockSpec((3 * D, D), lambda i: (0, 0)),
        pl.BlockSpec((1, D), lambda i: (0, 0)),
        pl.BlockSpec((D, OUT), lambda i: (0, 0)),
        pl.BlockSpec((1, OUT), lambda i: (0, 0)),
    ],
    out_shape=jax.ShapeDtypeStruct((1, OUT), jnp.float32),
    out_specs=pl.BlockSpec((1, OUT), lambda i: (0, 0)),
    scratch_shapes=[pltpu.VMEM((1, D), jnp.float32)],
)


def kernel(x, edge_index, edge_attr, batch, edge_batch,
           w1_0, b1_0, w2_0, b2_0, eps_0,
           w1_1, b1_1, w2_1, b2_1, eps_1,
           w1_2, b1_2, w2_2, b2_2, eps_2,
           lin1_w, lin1_b, lin2_w, lin2_b):
    ei = edge_index.reshape(2 * E)
    batchf = batch.astype(jnp.float32).reshape(N, 1)
    w = _tc_weights(batchf)

    layers = [(w1_0, b1_0, w2_0, b2_0, eps_0),
              (w1_1, b1_1, w2_1, b2_1, eps_1),
              (w1_2, b1_2, w2_2, b2_2, eps_2)]

    h = x
    pooled = []
    for l, (w1, b1, w2, b2, eps) in enumerate(layers[:2]):
        agg = _make_sc_aggregate()(h, ei).reshape(2, N, D)
        scale = (1.0 + eps).reshape(1, 1)
        h, p = _tc_layer(scale, h, agg, w,
                         w1, b1.reshape(1, D), w2, b2.reshape(1, D))
        pooled.append(p)

    (w1, b1, w2, b2, eps) = layers[2]
    agg = _make_sc_aggregate()(h, ei).reshape(2, N, D)
    scale = (1.0 + eps).reshape(1, 1)
    return _tc_last(scale, h, agg, w,
                    w1, b1.reshape(1, D), w2, b2.reshape(1, D),
                    pooled[0], pooled[1],
                    lin1_w, lin1_b.reshape(1, D),
                    lin2_w, lin2_b.reshape(1, OUT))


# speedup vs baseline: 1.2035x; 1.2035x over previous
import functools

import jax
import jax.numpy as jnp
from jax import lax
from jax.experimental import pallas as pl
from jax.experimental.pallas import tpu as pltpu
from jax.experimental.pallas import tpu_sc as plsc

N = 10000
E = 320000
D = 128
G = 64
OUT = 64

NC = 2
NS = 16
NW = NC * NS
E_PER_TILE = E // NW
CHUNK = 80
NCHUNK = E_PER_TILE // CHUNK
RBLK = 80
NRBLK = N // RBLK

NBUF = 4
IBUF = 2 * NBUF
K = 4
L = IBUF - NBUF + K
ZROWS = 16
GROUPS = NCHUNK // IBUF
TAIL = NCHUNK - GROUPS * IBUF


@functools.cache
def _make_sc_aggregate():
    mesh = plsc.VectorSubcoreMesh(core_axis_name="c", subcore_axis_name="s",
                                  num_cores=NC, num_subcores=NS)
    return pl.kernel(
        _sc_aggregate_body,
        out_type=jax.ShapeDtypeStruct((2 * N, D), jnp.float32),
        mesh=mesh,
        scratch_types=[
            pltpu.VMEM((ZROWS, D), jnp.float32),
            pltpu.VMEM_SHARED((N, D), jnp.float32),
        ]
        + [pltpu.VMEM((CHUNK, D), jnp.float32) for _ in range(NBUF)]
        + [pltpu.VMEM((CHUNK,), jnp.int32) for _ in range(IBUF)]
        + [pltpu.VMEM((CHUNK,), jnp.int32) for _ in range(IBUF)]
        + [pltpu.SemaphoreType.DMA for _ in range(2 * NBUF + IBUF)],
    )


def _sc_aggregate_body(h_hbm, ei_hbm, out_hbm, zbuf, acc, *rest):
    rows = rest[:NBUF]
    isl_s = rest[NBUF:NBUF + IBUF]
    isl_d = rest[NBUF + IBUF:NBUF + 2 * IBUF]
    sem_g = rest[NBUF + 2 * IBUF:2 * NBUF + 2 * IBUF]
    sem_s = rest[2 * NBUF + 2 * IBUF:3 * NBUF + 2 * IBUF]
    sem_i = rest[3 * NBUF + 2 * IBUF:]
    cid = lax.axis_index("c")
    sid = lax.axis_index("s")
    wid = sid * NC + cid
    base = wid * E_PER_TILE

    def load_idx(j, q):
        pltpu.async_copy(ei_hbm.at[pl.ds(base + j * CHUNK, CHUNK)],
                         isl_s[q], sem_i[q])
        pltpu.async_copy(ei_hbm.at[pl.ds(E + base + j * CHUNK, CHUNK)],
                         isl_d[q], sem_i[q])

    def wait_idx(j, q):
        pltpu.make_async_copy(ei_hbm.at[pl.ds(base + j * CHUNK, CHUNK)],
                              isl_s[q], sem_i[q]).wait()
        pltpu.make_async_copy(ei_hbm.at[pl.ds(E + base + j * CHUNK, CHUNK)],
                              isl_d[q], sem_i[q]).wait()

    def gather(b, q):
        pltpu.async_copy(h_hbm.at[isl_s[q]], rows[b], sem_g[b])

    def wait_gather(b, q):
        pltpu.make_async_copy(h_hbm.at[isl_s[q]], rows[b], sem_g[b]).wait()

    def scatter(b, q):
        pltpu.async_copy(rows[b], acc.at[isl_d[q]], sem_s[b], add=True)

    def wait_scatter(b, q):
        pltpu.make_async_copy(rows[b], acc.at[isl_d[q]], sem_s[b]).wait()

    for q in range(L):
        load_idx(q, q)

    z16 = jnp.zeros((16,), jnp.float32)

    def _zrow(i, carry):
        for j in range(D // 16):
            zbuf[i, pl.ds(j * 16, 16)] = z16
        return carry

    lax.fori_loop(0, ZROWS, _zrow, 0)
    nzblk = N // ZROWS

    def _zblk(k, carry):
        blk = sid + k * NS

        @pl.when(blk < nzblk)
        def _():
            pltpu.sync_copy(zbuf, acc.at[pl.ds(blk * ZROWS, ZROWS)])
        return carry

    lax.fori_loop(0, pl.cdiv(nzblk, NS), _zblk, 0)

    for c in range(K):
        wait_idx(c, c)
        gather(c % NBUF, c)
    plsc.subcore_barrier()

    def _step(j, u, static):
        b = u % NBUF
        wait_gather(b, u)
        scatter(b, u)
        b2 = (u + K) % NBUF
        u2 = (u + K) % IBUF
        ud = (u + K - NBUF) % IBUF

        def _drain_prev():
            wait_scatter(b2, ud)

        def _prefetch():
            wait_idx(j + K, u2)
            gather(b2, u2)

        def _load():
            load_idx(j + L, (u + L) % IBUF)

        if static:
            if j + K < NCHUNK:
                if j >= NBUF - K:
                    _drain_prev()
                _prefetch()
            if j + L < NCHUNK:
                _load()
        else:
            def _both():
                pl.when(j >= NBUF - K)(_drain_prev)
                _prefetch()

            pl.when(j + K < NCHUNK)(_both)
            pl.when(j + L < NCHUNK)(_load)

    def _group(g, carry):
        for u in range(IBUF):
            _step(g * IBUF + u, u, False)
        return carry

    lax.fori_loop(0, GROUPS, _group, 0)

    for t in range(TAIL):
        j = GROUPS * IBUF + t
        _step(j, j % IBUF, True)

    for c in range(NCHUNK - NBUF, NCHUNK):
        wait_scatter(c % NBUF, c % IBUF)
    plsc.subcore_barrier()

    def _oblk(k, carry):
        blk = sid + k * NS

        @pl.when(blk < NRBLK)
        def _():
            pltpu.sync_copy(acc.at[pl.ds(blk * RBLK, RBLK)],
                            out_hbm.at[pl.ds(cid * N + blk * RBLK, RBLK)])
        return carry

    lax.fori_loop(0, pl.cdiv(NRBLK, NS), _oblk, 0)


BR = 2000
NB = N // BR


_PREC = lax.Precision.DEFAULT


def _tc_weights_body(batchf_ref, w_ref):
    onehot = (batchf_ref[...] ==
              lax.broadcasted_iota(jnp.int32, (N, G), 1).astype(jnp.float32)
              ).astype(jnp.float32)
    cnt = jnp.sum(onehot, axis=0, keepdims=True)
    inv = 1.0 / jnp.maximum(cnt, 1.0)
    w_ref[...] = lax.dot_general(onehot, inv, (((1,), (1,)), ((), ())),
                                 preferred_element_type=jnp.float32,
                                 precision=_PREC)


_tc_weights = pl.pallas_call(
    _tc_weights_body,
    out_shape=jax.ShapeDtypeStruct((N, 1), jnp.float32),
)


def _mlp(h, agg0, agg1, scale, w1, b1, w2, b2):
    u = h * scale + agg0 + agg1
    t = jnp.maximum(jnp.dot(u, w1, preferred_element_type=jnp.float32,
                            precision=_PREC) + b1, 0.0)
    return jnp.dot(t, w2, preferred_element_type=jnp.float32,
                   precision=_PREC) + b2


def _tc_layer_body(scale_ref, h_ref, agg_ref, w_ref,
                   w1_ref, b1_ref, w2_ref, b2_ref, out_h_ref, out_p_ref):
    i = pl.program_id(0)
    v = _mlp(h_ref[...], agg_ref[0], agg_ref[1], scale_ref[0, 0],
             w1_ref[...], b1_ref[...], w2_ref[...], b2_ref[...])
    out_h_ref[...] = v
    part = lax.dot_general(w_ref[...], v, (((0,), (0,)), ((), ())),
                           preferred_element_type=jnp.float32,
                           precision=_PREC)

    @pl.when(i == 0)
    def _():
        out_p_ref[...] = part

    @pl.when(i != 0)
    def _():
        out_p_ref[...] += part


_layer_in_specs = [
    pl.BlockSpec((1, 1), lambda i: (0, 0)),
    pl.BlockSpec((BR, D), lambda i: (i, 0)),
    pl.BlockSpec((2, BR, D), lambda i: (0, i, 0)),
    pl.BlockSpec((BR, 1), lambda i: (i, 0)),
    pl.BlockSpec((D, D), lambda i: (0, 0)),
    pl.BlockSpec((1, D), lambda i: (0, 0)),
    pl.BlockSpec((D, D), lambda i: (0, 0)),
    pl.BlockSpec((1, D), lambda i: (0, 0)),
]

_tc_layer = pl.pallas_call(
    _tc_layer_body,
    grid=(NB,),
    in_specs=_layer_in_specs,
    out_shape=[jax.ShapeDtypeStruct((N, D), jnp.float32),
               jax.ShapeDtypeStruct((1, D), jnp.float32)],
    out_specs=[pl.BlockSpec((BR, D), lambda i: (i, 0)),
               pl.BlockSpec((1, D), lambda i: (0, 0))],
)


def _tc_last_body(scale_ref, h_ref, agg_ref, w_ref,
                  w1_ref, b1_ref, w2_ref, b2_ref,
                  p1_ref, p2_ref, l1w_ref, l1b_ref, l2w_ref, l2b_ref,
                  out_ref, pacc_ref):
    i = pl.program_id(0)
    v = _mlp(h_ref[...], agg_ref[0], agg_ref[1], scale_ref[0, 0],
             w1_ref[...], b1_ref[...], w2_ref[...], b2_ref[...])
    part = lax.dot_general(w_ref[...], v, (((0,), (0,)), ((), ())),
                           preferred_element_type=jnp.float32,
                           precision=_PREC)

    @pl.when(i == 0)
    def _():
        pacc_ref[...] = part

    @pl.when(i != 0)
    def _():
        pacc_ref[...] += part

    @pl.when(i == NB - 1)
    def _():
        p = jnp.concatenate([p1_ref[...], p2_ref[...], pacc_ref[...]], axis=1)
        o = jnp.maximum(jnp.dot(p, l1w_ref[...],
                                preferred_element_type=jnp.float32,
                                precision=_PREC) + l1b_ref[...], 0.0)
        o = jnp.maximum(jnp.dot(o, l2w_ref[...],
                                preferred_element_type=jnp.float32,
                                precision=_PREC) + l2b_ref[...], 0.0)
        out_ref[...] = o


_tc_last = pl.pallas_call(
    _tc_last_body,
    grid=(NB,),
    in_specs=_layer_in_specs + [
        pl.BlockSpec((1, D), lambda i: (0, 0)),
        pl.BlockSpec((1, D), lambda i: (0, 0)),
        pl.BlockSpec((3 * D, D), lambda i: (0, 0)),
        pl.BlockSpec((1, D), lambda i: (0, 0)),
        pl.BlockSpec((D, OUT), lambda i: (0, 0)),
        pl.BlockSpec((1, OUT), lambda i: (0, 0)),
    ],
    out_shape=jax.ShapeDtypeStruct((1, OUT), jnp.float32),
    out_specs=pl.BlockSpec((1, OUT), lambda i: (0, 0)),
    scratch_shapes=[pltpu.VMEM((1, D), jnp.float32)],
)


def kernel(x, edge_index, edge_attr, batch, edge_batch,
           w1_0, b1_0, w2_0, b2_0, eps_0,
           w1_1, b1_1, w2_1, b2_1, eps_1,
           w1_2, b1_2, w2_2, b2_2, eps_2,
           lin1_w, lin1_b, lin2_w, lin2_b):
    ei = edge_index.reshape(2 * E)
    batchf = batch.astype(jnp.float32).reshape(N, 1)
    w = _tc_weights(batchf)

    layers = [(w1_0, b1_0, w2_0, b2_0, eps_0),
              (w1_1, b1_1, w2_1, b2_1, eps_1),
              (w1_2, b1_2, w2_2, b2_2, eps_2)]

    h = x
    pooled = []
    for l, (w1, b1, w2, b2, eps) in enumerate(layers[:2]):
        agg = _make_sc_aggregate()(h, ei).reshape(2, N, D)
        scale = (1.0 + eps).reshape(1, 1)
        h, p = _tc_layer(scale, h, agg, w,
                         w1, b1.reshape(1, D), w2, b2.reshape(1, D))
        pooled.append(p)

    (w1, b1, w2, b2, eps) = layers[2]
    agg = _make_sc_aggregate()(h, ei).reshape(2, N, D)
    scale = (1.0 + eps).reshape(1, 1)
    return _tc_last(scale, h, agg, w,
                    w1, b1.reshape(1, D), w2, b2.reshape(1, D),
                    pooled[0], pooled[1],
                    lin1_w, lin1_b.reshape(1, D),
                    lin2_w, lin2_b.reshape(1, OUT))


# speedup vs baseline: 1.2437x; 1.0334x over previous
import functools

import jax
import jax.numpy as jnp
from jax import lax
from jax.experimental import pallas as pl
from jax.experimental.pallas import tpu as pltpu
from jax.experimental.pallas import tpu_sc as plsc

N = 10000
E = 320000
D = 128
G = 64
OUT = 64

NC = 2
NS = 16
NW = NC * NS
E_PER_TILE = E // NW
CHUNK = 80
NCHUNK = E_PER_TILE // CHUNK
RBLK = 80
NRBLK = N // RBLK

NBUF = 4
IBUF = 2 * NBUF
K = 4
L = IBUF - NBUF + K
ZROWS = 40
GROUPS = NCHUNK // IBUF
TAIL = NCHUNK - GROUPS * IBUF


@functools.cache
def _make_sc_aggregate():
    mesh = plsc.VectorSubcoreMesh(core_axis_name="c", subcore_axis_name="s",
                                  num_cores=NC, num_subcores=NS)
    return pl.kernel(
        _sc_aggregate_body,
        out_type=jax.ShapeDtypeStruct((2 * N, D), jnp.float32),
        mesh=mesh,
        scratch_types=[
            pltpu.VMEM((ZROWS, D), jnp.float32),
            pltpu.VMEM_SHARED((N, D), jnp.float32),
        ]
        + [pltpu.VMEM((CHUNK, D), jnp.float32) for _ in range(NBUF)]
        + [pltpu.VMEM((CHUNK,), jnp.int32) for _ in range(IBUF)]
        + [pltpu.VMEM((CHUNK,), jnp.int32) for _ in range(IBUF)]
        + [pltpu.SemaphoreType.DMA for _ in range(2 * NBUF + IBUF)],
    )


def _sc_aggregate_body(h_hbm, ei_hbm, out_hbm, zbuf, acc, *rest):
    rows = rest[:NBUF]
    isl_s = rest[NBUF:NBUF + IBUF]
    isl_d = rest[NBUF + IBUF:NBUF + 2 * IBUF]
    sem_g = rest[NBUF + 2 * IBUF:2 * NBUF + 2 * IBUF]
    sem_s = rest[2 * NBUF + 2 * IBUF:3 * NBUF + 2 * IBUF]
    sem_i = rest[3 * NBUF + 2 * IBUF:]
    cid = lax.axis_index("c")
    sid = lax.axis_index("s")
    wid = sid * NC + cid
    base = wid * E_PER_TILE

    def load_idx(j, q):
        pltpu.async_copy(ei_hbm.at[pl.ds(base + j * CHUNK, CHUNK)],
                         isl_s[q], sem_i[q])
        pltpu.async_copy(ei_hbm.at[pl.ds(E + base + j * CHUNK, CHUNK)],
                         isl_d[q], sem_i[q])

    def wait_idx(j, q):
        pltpu.make_async_copy(ei_hbm.at[pl.ds(base + j * CHUNK, CHUNK)],
                              isl_s[q], sem_i[q]).wait()
        pltpu.make_async_copy(ei_hbm.at[pl.ds(E + base + j * CHUNK, CHUNK)],
                              isl_d[q], sem_i[q]).wait()

    def gather(b, q):
        pltpu.async_copy(h_hbm.at[isl_s[q]], rows[b], sem_g[b])

    def wait_gather(b, q):
        pltpu.make_async_copy(h_hbm.at[isl_s[q]], rows[b], sem_g[b]).wait()

    def scatter(b, q):
        pltpu.async_copy(rows[b], acc.at[isl_d[q]], sem_s[b], add=True)

    def wait_scatter(b, q):
        pltpu.make_async_copy(rows[b], acc.at[isl_d[q]], sem_s[b]).wait()

    for q in range(L):
        load_idx(q, q)

    z16 = jnp.zeros((16,), jnp.float32)

    def _zrow(i, carry):
        for j in range(D // 16):
            zbuf[i, pl.ds(j * 16, 16)] = z16
        return carry

    lax.fori_loop(0, ZROWS, _zrow, 0)

    for c in range(K):
        wait_idx(c, c)
        gather(c % NBUF, c)

    nzblk = N // ZROWS

    def _zblk(k, carry):
        blk = sid + k * NS

        @pl.when(blk < nzblk)
        def _():
            pltpu.sync_copy(zbuf, acc.at[pl.ds(blk * ZROWS, ZROWS)])
        return carry

    lax.fori_loop(0, pl.cdiv(nzblk, NS), _zblk, 0)
    plsc.subcore_barrier()

    def _step(j, u, static):
        b = u % NBUF
        wait_gather(b, u)
        scatter(b, u)
        b2 = (u + K) % NBUF
        u2 = (u + K) % IBUF
        ud = (u + K - NBUF) % IBUF

        def _drain_prev():
            wait_scatter(b2, ud)

        def _prefetch():
            wait_idx(j + K, u2)
            gather(b2, u2)

        def _load():
            load_idx(j + L, (u + L) % IBUF)

        if static:
            if j + K < NCHUNK:
                if j >= NBUF - K:
                    _drain_prev()
                _prefetch()
            if j + L < NCHUNK:
                _load()
        else:
            def _both():
                pl.when(j >= NBUF - K)(_drain_prev)
                _prefetch()

            pl.when(j + K < NCHUNK)(_both)
            pl.when(j + L < NCHUNK)(_load)

    def _group(g, carry):
        for u in range(IBUF):
            _step(g * IBUF + u, u, False)
        return carry

    lax.fori_loop(0, GROUPS, _group, 0)

    for t in range(TAIL):
        j = GROUPS * IBUF + t
        _step(j, j % IBUF, True)

    for c in range(NCHUNK - NBUF, NCHUNK):
        wait_scatter(c % NBUF, c % IBUF)
    plsc.subcore_barrier()

    def _oblk(k, carry):
        blk = sid + k * NS

        @pl.when(blk < NRBLK)
        def _():
            pltpu.sync_copy(acc.at[pl.ds(blk * RBLK, RBLK)],
                            out_hbm.at[pl.ds(cid * N + blk * RBLK, RBLK)])
        return carry

    lax.fori_loop(0, pl.cdiv(NRBLK, NS), _oblk, 0)


BR = 2000
NB = N // BR


_PREC = lax.Precision.DEFAULT


def _tc_weights_body(batchf_ref, w_ref):
    onehot = (batchf_ref[...] ==
              lax.broadcasted_iota(jnp.int32, (N, G), 1).astype(jnp.float32)
              ).astype(jnp.float32)
    cnt = jnp.sum(onehot, axis=0, keepdims=True)
    inv = 1.0 / jnp.maximum(cnt, 1.0)
    w_ref[...] = lax.dot_general(onehot, inv, (((1,), (1,)), ((), ())),
                                 preferred_element_type=jnp.float32,
                                 precision=_PREC)


_tc_weights = pl.pallas_call(
    _tc_weights_body,
    out_shape=jax.ShapeDtypeStruct((N, 1), jnp.float32),
)


def _mlp(h, agg0, agg1, scale, w1, b1, w2, b2):
    u = h * scale + agg0 + agg1
    t = jnp.maximum(jnp.dot(u, w1, preferred_element_type=jnp.float32,
                            precision=_PREC) + b1, 0.0)
    return jnp.dot(t, w2, preferred_element_type=jnp.float32,
                   precision=_PREC) + b2


def _tc_layer_body(scale_ref, h_ref, agg_ref, w_ref,
                   w1_ref, b1_ref, w2_ref, b2_ref, out_h_ref, out_p_ref):
    i = pl.program_id(0)
    v = _mlp(h_ref[...], agg_ref[0], agg_ref[1], scale_ref[0, 0],
             w1_ref[...], b1_ref[...], w2_ref[...], b2_ref[...])
    out_h_ref[...] = v
    part = lax.dot_general(w_ref[...], v, (((0,), (0,)), ((), ())),
                           preferred_element_type=jnp.float32,
                           precision=_PREC)

    @pl.when(i == 0)
    def _():
        out_p_ref[...] = part

    @pl.when(i != 0)
    def _():
        out_p_ref[...] += part


_layer_in_specs = [
    pl.BlockSpec((1, 1), lambda i: (0, 0)),
    pl.BlockSpec((BR, D), lambda i: (i, 0)),
    pl.BlockSpec((2, BR, D), lambda i: (0, i, 0)),
    pl.BlockSpec((BR, 1), lambda i: (i, 0)),
    pl.BlockSpec((D, D), lambda i: (0, 0)),
    pl.BlockSpec((1, D), lambda i: (0, 0)),
    pl.BlockSpec((D, D), lambda i: (0, 0)),
    pl.BlockSpec((1, D), lambda i: (0, 0)),
]

_tc_layer = pl.pallas_call(
    _tc_layer_body,
    grid=(NB,),
    in_specs=_layer_in_specs,
    out_shape=[jax.ShapeDtypeStruct((N, D), jnp.float32),
               jax.ShapeDtypeStruct((1, D), jnp.float32)],
    out_specs=[pl.BlockSpec((BR, D), lambda i: (i, 0)),
               pl.BlockSpec((1, D), lambda i: (0, 0))],
)


def _tc_last_body(scale_ref, h_ref, agg_ref, w_ref,
                  w1_ref, b1_ref, w2_ref, b2_ref,
                  p1_ref, p2_ref, l1w_ref, l1b_ref, l2w_ref, l2b_ref,
                  out_ref, pacc_ref):
    i = pl.program_id(0)
    v = _mlp(h_ref[...], agg_ref[0], agg_ref[1], scale_ref[0, 0],
             w1_ref[...], b1_ref[...], w2_ref[...], b2_ref[...])
    part = lax.dot_general(w_ref[...], v, (((0,), (0,)), ((), ())),
                           preferred_element_type=jnp.float32,
                           precision=_PREC)

    @pl.when(i == 0)
    def _():
        pacc_ref[...] = part

    @pl.when(i != 0)
    def _():
        pacc_ref[...] += part

    @pl.when(i == NB - 1)
    def _():
        p = jnp.concatenate([p1_ref[...], p2_ref[...], pacc_ref[...]], axis=1)
        o = jnp.maximum(jnp.dot(p, l1w_ref[...],
                                preferred_element_type=jnp.float32,
                                precision=_PREC) + l1b_ref[...], 0.0)
        o = jnp.maximum(jnp.dot(o, l2w_ref[...],
                                preferred_element_type=jnp.float32,
                                precision=_PREC) + l2b_ref[...], 0.0)
        out_ref[...] = o


_tc_last = pl.pallas_call(
    _tc_last_body,
    grid=(NB,),
    in_specs=_layer_in_specs + [
        pl.BlockSpec((1, D), lambda i: (0, 0)),
        pl.BlockSpec((1, D), lambda i: (0, 0)),
        pl.BlockSpec((3 * D, D), lambda i: (0, 0)),
        pl.BlockSpec((1, D), lambda i: (0, 0)),
        pl.BlockSpec((D, OUT), lambda i: (0, 0)),
        pl.BlockSpec((1, OUT), lambda i: (0, 0)),
    ],
    out_shape=jax.ShapeDtypeStruct((1, OUT), jnp.float32),
    out_specs=pl.BlockSpec((1, OUT), lambda i: (0, 0)),
    scratch_shapes=[pltpu.VMEM((1, D), jnp.float32)],
)


def kernel(x, edge_index, edge_attr, batch, edge_batch,
           w1_0, b1_0, w2_0, b2_0, eps_0,
           w1_1, b1_1, w2_1, b2_1, eps_1,
           w1_2, b1_2, w2_2, b2_2, eps_2,
           lin1_w, lin1_b, lin2_w, lin2_b):
    ei = edge_index.reshape(2 * E)
    batchf = batch.astype(jnp.float32).reshape(N, 1)
    w = _tc_weights(batchf)

    layers = [(w1_0, b1_0, w2_0, b2_0, eps_0),
              (w1_1, b1_1, w2_1, b2_1, eps_1),
              (w1_2, b1_2, w2_2, b2_2, eps_2)]

    h = x
    pooled = []
    for l, (w1, b1, w2, b2, eps) in enumerate(layers[:2]):
        agg = _make_sc_aggregate()(h, ei).reshape(2, N, D)
        scale = (1.0 + eps).reshape(1, 1)
        h, p = _tc_layer(scale, h, agg, w,
                         w1, b1.reshape(1, D), w2, b2.reshape(1, D))
        pooled.append(p)

    (w1, b1, w2, b2, eps) = layers[2]
    agg = _make_sc_aggregate()(h, ei).reshape(2, N, D)
    scale = (1.0 + eps).reshape(1, 1)
    return _tc_last(scale, h, agg, w,
                    w1, b1.reshape(1, D), w2, b2.reshape(1, D),
                    pooled[0], pooled[1],
                    lin1_w, lin1_b.reshape(1, D),
                    lin2_w, lin2_b.reshape(1, OUT))


# speedup vs baseline: 1.2448x; 1.0009x over previous
import functools

import jax
import jax.numpy as jnp
from jax import lax
from jax.experimental import pallas as pl
from jax.experimental.pallas import tpu as pltpu
from jax.experimental.pallas import tpu_sc as plsc

N = 10000
E = 320000
D = 128
G = 64
OUT = 64

NC = 2
NS = 16
NW = NC * NS
E_PER_TILE = E // NW
CHUNK = 80
NCHUNK = E_PER_TILE // CHUNK
RBLK = 80
NRBLK = N // RBLK

NBUF = 4
IBUF = 2 * NBUF
K = 4
L = IBUF - NBUF + K
ZROWS = 40
GROUPS = NCHUNK // IBUF
TAIL = NCHUNK - GROUPS * IBUF


@functools.cache
def _make_sc_aggregate():
    mesh = plsc.VectorSubcoreMesh(core_axis_name="c", subcore_axis_name="s",
                                  num_cores=NC, num_subcores=NS)
    return pl.kernel(
        _sc_aggregate_body,
        out_type=jax.ShapeDtypeStruct((2 * N, D), jnp.float32),
        mesh=mesh,
        scratch_types=[
            pltpu.VMEM((ZROWS, D), jnp.float32),
            pltpu.VMEM_SHARED((N, D), jnp.float32),
        ]
        + [pltpu.VMEM((CHUNK, D), jnp.float32) for _ in range(NBUF)]
        + [pltpu.VMEM((CHUNK,), jnp.int32) for _ in range(IBUF)]
        + [pltpu.VMEM((CHUNK,), jnp.int32) for _ in range(IBUF)]
        + [pltpu.SemaphoreType.DMA for _ in range(2 * NBUF + IBUF + 1)],
    )


def _sc_aggregate_body(h_hbm, ei_hbm, out_hbm, zbuf, acc, *rest):
    rows = rest[:NBUF]
    isl_s = rest[NBUF:NBUF + IBUF]
    isl_d = rest[NBUF + IBUF:NBUF + 2 * IBUF]
    sem_g = rest[NBUF + 2 * IBUF:2 * NBUF + 2 * IBUF]
    sem_s = rest[2 * NBUF + 2 * IBUF:3 * NBUF + 2 * IBUF]
    sem_i = rest[3 * NBUF + 2 * IBUF:3 * NBUF + 3 * IBUF]
    sem_o = rest[3 * NBUF + 3 * IBUF]
    cid = lax.axis_index("c")
    sid = lax.axis_index("s")
    wid = sid * NC + cid
    base = wid * E_PER_TILE

    def load_idx(j, q):
        pltpu.async_copy(ei_hbm.at[pl.ds(base + j * CHUNK, CHUNK)],
                         isl_s[q], sem_i[q])
        pltpu.async_copy(ei_hbm.at[pl.ds(E + base + j * CHUNK, CHUNK)],
                         isl_d[q], sem_i[q])

    def wait_idx(j, q):
        pltpu.make_async_copy(ei_hbm.at[pl.ds(base + j * CHUNK, CHUNK)],
                              isl_s[q], sem_i[q]).wait()
        pltpu.make_async_copy(ei_hbm.at[pl.ds(E + base + j * CHUNK, CHUNK)],
                              isl_d[q], sem_i[q]).wait()

    def gather(b, q):
        pltpu.async_copy(h_hbm.at[isl_s[q]], rows[b], sem_g[b])

    def wait_gather(b, q):
        pltpu.make_async_copy(h_hbm.at[isl_s[q]], rows[b], sem_g[b]).wait()

    def scatter(b, q):
        pltpu.async_copy(rows[b], acc.at[isl_d[q]], sem_s[b], add=True)

    def wait_scatter(b, q):
        pltpu.make_async_copy(rows[b], acc.at[isl_d[q]], sem_s[b]).wait()

    for q in range(L):
        load_idx(q, q)

    z16 = jnp.zeros((16,), jnp.float32)

    def _zrow(i, carry):
        for j in range(D // 16):
            zbuf[i, pl.ds(j * 16, 16)] = z16
        return carry

    lax.fori_loop(0, ZROWS, _zrow, 0)

    for c in range(K):
        wait_idx(c, c)
        gather(c % NBUF, c)

    nzblk = N // ZROWS

    def _zblk(k, carry):
        blk = sid + k * NS

        @pl.when(blk < nzblk)
        def _():
            pltpu.async_copy(zbuf, acc.at[pl.ds(blk * ZROWS, ZROWS)], sem_o)
        return carry

    lax.fori_loop(0, pl.cdiv(nzblk, NS), _zblk, 0)

    def _zdrain(k, carry):
        blk = sid + k * NS

        @pl.when(blk < nzblk)
        def _():
            pltpu.make_async_copy(zbuf, acc.at[pl.ds(blk * ZROWS, ZROWS)],
                                  sem_o).wait()
        return carry

    lax.fori_loop(0, pl.cdiv(nzblk, NS), _zdrain, 0)
    plsc.subcore_barrier()

    def _step(j, u, static):
        b = u % NBUF
        wait_gather(b, u)
        scatter(b, u)
        b2 = (u + K) % NBUF
        u2 = (u + K) % IBUF
        ud = (u + K - NBUF) % IBUF

        def _drain_prev():
            wait_scatter(b2, ud)

        def _prefetch():
            wait_idx(j + K, u2)
            gather(b2, u2)

        def _load():
            load_idx(j + L, (u + L) % IBUF)

        if static:
            if j + K < NCHUNK:
                if j >= NBUF - K:
                    _drain_prev()
                _prefetch()
            if j + L < NCHUNK:
                _load()
        else:
            def _both():
                pl.when(j >= NBUF - K)(_drain_prev)
                _prefetch()

            pl.when(j + K < NCHUNK)(_both)
            pl.when(j + L < NCHUNK)(_load)

    def _group(g, carry):
        for u in range(IBUF):
            _step(g * IBUF + u, u, False)
        return carry

    lax.fori_loop(0, GROUPS, _group, 0)

    for t in range(TAIL):
        j = GROUPS * IBUF + t
        _step(j, j % IBUF, True)

    for c in range(NCHUNK - NBUF, NCHUNK):
        wait_scatter(c % NBUF, c % IBUF)
    plsc.subcore_barrier()

    def _oblk(k, carry):
        blk = sid + k * NS

        @pl.when(blk < NRBLK)
        def _():
            pltpu.async_copy(acc.at[pl.ds(blk * RBLK, RBLK)],
                             out_hbm.at[pl.ds(cid * N + blk * RBLK, RBLK)],
                             sem_o)
        return carry

    lax.fori_loop(0, pl.cdiv(NRBLK, NS), _oblk, 0)

    def _odrain(k, carry):
        blk = sid + k * NS

        @pl.when(blk < NRBLK)
        def _():
            pltpu.make_async_copy(
                acc.at[pl.ds(blk * RBLK, RBLK)],
                out_hbm.at[pl.ds(cid * N + blk * RBLK, RBLK)],
                sem_o).wait()
        return carry

    lax.fori_loop(0, pl.cdiv(NRBLK, NS), _odrain, 0)


BR = 2000
NB = N // BR


_PREC = lax.Precision.DEFAULT


def _tc_weights_body(batchf_ref, w_ref):
    onehot = (batchf_ref[...] ==
              lax.broadcasted_iota(jnp.int32, (N, G), 1).astype(jnp.float32)
              ).astype(jnp.float32)
    cnt = jnp.sum(onehot, axis=0, keepdims=True)
    inv = 1.0 / jnp.maximum(cnt, 1.0)
    w_ref[...] = lax.dot_general(onehot, inv, (((1,), (1,)), ((), ())),
                                 preferred_element_type=jnp.float32,
                                 precision=_PREC)


_tc_weights = pl.pallas_call(
    _tc_weights_body,
    out_shape=jax.ShapeDtypeStruct((N, 1), jnp.float32),
)


def _mlp(h, agg0, agg1, scale, w1, b1, w2, b2):
    u = h * scale + agg0 + agg1
    t = jnp.maximum(jnp.dot(u, w1, preferred_element_type=jnp.float32,
                            precision=_PREC) + b1, 0.0)
    return jnp.dot(t, w2, preferred_element_type=jnp.float32,
                   precision=_PREC) + b2


def _tc_layer_body(scale_ref, h_ref, agg_ref, w_ref,
                   w1_ref, b1_ref, w2_ref, b2_ref, out_h_ref, out_p_ref):
    i = pl.program_id(0)
    v = _mlp(h_ref[...], agg_ref[0], agg_ref[1], scale_ref[0, 0],
             w1_ref[...], b1_ref[...], w2_ref[...], b2_ref[...])
    out_h_ref[...] = v
    part = lax.dot_general(w_ref[...], v, (((0,), (0,)), ((), ())),
                           preferred_element_type=jnp.float32,
                           precision=_PREC)

    @pl.when(i == 0)
    def _():
        out_p_ref[...] = part

    @pl.when(i != 0)
    def _():
        out_p_ref[...] += part


_layer_in_specs = [
    pl.BlockSpec((1, 1), lambda i: (0, 0)),
    pl.BlockSpec((BR, D), lambda i: (i, 0)),
    pl.BlockSpec((2, BR, D), lambda i: (0, i, 0)),
    pl.BlockSpec((BR, 1), lambda i: (i, 0)),
    pl.BlockSpec((D, D), lambda i: (0, 0)),
    pl.BlockSpec((1, D), lambda i: (0, 0)),
    pl.BlockSpec((D, D), lambda i: (0, 0)),
    pl.BlockSpec((1, D), lambda i: (0, 0)),
]

_tc_layer = pl.pallas_call(
    _tc_layer_body,
    grid=(NB,),
    in_specs=_layer_in_specs,
    out_shape=[jax.ShapeDtypeStruct((N, D), jnp.float32),
               jax.ShapeDtypeStruct((1, D), jnp.float32)],
    out_specs=[pl.BlockSpec((BR, D), lambda i: (i, 0)),
               pl.BlockSpec((1, D), lambda i: (0, 0))],
)


def _tc_last_body(scale_ref, h_ref, agg_ref, w_ref,
                  w1_ref, b1_ref, w2_ref, b2_ref,
                  p1_ref, p2_ref, l1w_ref, l1b_ref, l2w_ref, l2b_ref,
                  out_ref, pacc_ref):
    i = pl.program_id(0)
    v = _mlp(h_ref[...], agg_ref[0], agg_ref[1], scale_ref[0, 0],
             w1_ref[...], b1_ref[...], w2_ref[...], b2_ref[...])
    part = lax.dot_general(w_ref[...], v, (((0,), (0,)), ((), ())),
                           preferred_element_type=jnp.float32,
                           precision=_PREC)

    @pl.when(i == 0)
    def _():
        pacc_ref[...] = part

    @pl.when(i != 0)
    def _():
        pacc_ref[...] += part

    @pl.when(i == NB - 1)
    def _():
        p = jnp.concatenate([p1_ref[...], p2_ref[...], pacc_ref[...]], axis=1)
        o = jnp.maximum(jnp.dot(p, l1w_ref[...],
                                preferred_element_type=jnp.float32,
                                precision=_PREC) + l1b_ref[...], 0.0)
        o = jnp.maximum(jnp.dot(o, l2w_ref[...],
                                preferred_element_type=jnp.float32,
                                precision=_PREC) + l2b_ref[...], 0.0)
        out_ref[...] = o


_tc_last = pl.pallas_call(
    _tc_last_body,
    grid=(NB,),
    in_specs=_layer_in_specs + [
        pl.BlockSpec((1, D), lambda i: (0, 0)),
        pl.BlockSpec((1, D), lambda i: (0, 0)),
        pl.BlockSpec((3 * D, D), lambda i: (0, 0)),
        pl.BlockSpec((1, D), lambda i: (0, 0)),
        pl.BlockSpec((D, OUT), lambda i: (0, 0)),
        pl.BlockSpec((1, OUT), lambda i: (0, 0)),
    ],
    out_shape=jax.ShapeDtypeStruct((1, OUT), jnp.float32),
    out_specs=pl.BlockSpec((1, OUT), lambda i: (0, 0)),
    scratch_shapes=[pltpu.VMEM((1, D), jnp.float32)],
)


def kernel(x, edge_index, edge_attr, batch, edge_batch,
           w1_0, b1_0, w2_0, b2_0, eps_0,
           w1_1, b1_1, w2_1, b2_1, eps_1,
           w1_2, b1_2, w2_2, b2_2, eps_2,
           lin1_w, lin1_b, lin2_w, lin2_b):
    ei = edge_index.reshape(2 * E)
    batchf = batch.astype(jnp.float32).reshape(N, 1)
    w = _tc_weights(batchf)

    layers = [(w1_0, b1_0, w2_0, b2_0, eps_0),
              (w1_1, b1_1, w2_1, b2_1, eps_1),
              (w1_2, b1_2, w2_2, b2_2, eps_2)]

    h = x
    pooled = []
    for l, (w1, b1, w2, b2, eps) in enumerate(layers[:2]):
        agg = _make_sc_aggregate()(h, ei).reshape(2, N, D)
        scale = (1.0 + eps).reshape(1, 1)
        h, p = _tc_layer(scale, h, agg, w,
                         w1, b1.reshape(1, D), w2, b2.reshape(1, D))
        pooled.append(p)

    (w1, b1, w2, b2, eps) = layers[2]
    agg = _make_sc_aggregate()(h, ei).reshape(2, N, D)
    scale = (1.0 + eps).reshape(1, 1)
    return _tc_last(scale, h, agg, w,
                    w1, b1.reshape(1, D), w2, b2.reshape(1, D),
                    pooled[0], pooled[1],
                    lin1_w, lin1_b.reshape(1, D),
                    lin2_w, lin2_b.reshape(1, OUT))
